# Initial kernel scaffold; baseline (speedup 1.0000x reference)
import functools

import jax
import jax.numpy as jnp
from jax import lax
from jax.experimental import pallas as pl
from jax.experimental.pallas import tpu as pltpu
from jax.experimental.pallas import tpu_sc as plsc

NC = 2
NS = 16
NW = NC * NS


def _sc_kernel(N, E, K, V, D, C, EPT):
    per_w = N // NW
    n_chunks = per_w // C
    E_pad = ((E + EPT - 1) // EPT) * EPT
    mesh = plsc.VectorSubcoreMesh(
        core_axis_name="c", subcore_axis_name="s",
        num_cores=NC, num_subcores=NS)

    @functools.partial(
        pl.kernel,
        mesh=mesh,
        out_type=[
            jax.ShapeDtypeStruct((N, D), jnp.float32),
            jax.ShapeDtypeStruct((E_pad, D), jnp.float32),
        ],
        scratch_types=[
            pltpu.VMEM((C,), jnp.int32),
            pltpu.VMEM((C, D), jnp.float32),
            pltpu.VMEM((EPT, D), jnp.float32),
            pltpu.SemaphoreType.DMA,
        ],
    )
    def k(ids_hbm, e2s_hbm, table_hbm, out_hbm, emb_hbm, idx_v, rows_v,
          emb_v, sem):
        c = lax.axis_index("c")
        s = lax.axis_index("s")
        wid = s * NC + c

        base = jnp.minimum(s * EPT, E - EPT)
        pltpu.sync_copy(e2s_hbm.at[pl.ds(base * K, EPT * K)], idx_v)
        pltpu.async_copy(table_hbm.at[idx_v], rows_v, sem).wait()

        def env_body(e, _):
            for d in range(D // 16):
                sl = pl.ds(d * 16, 16)
                acc = rows_v[e * K, sl]
                for j in range(1, K):
                    acc = acc + rows_v[e * K + j, sl]
                emb_v[e, sl] = acc
            return 0

        lax.fori_loop(0, EPT, env_body, 0)
        pltpu.sync_copy(emb_v, emb_hbm.at[pl.ds(base, EPT)])
        plsc.subcore_barrier()

        def chunk_body(i, _):
            off = wid * per_w + i * C
            pltpu.sync_copy(ids_hbm.at[pl.ds(off, C)], idx_v)
            pltpu.async_copy(emb_hbm.at[idx_v], rows_v, sem).wait()
            pltpu.sync_copy(rows_v, out_hbm.at[pl.ds(off, C)])
            return 0

        lax.fori_loop(0, n_chunks, chunk_body, 0)

    return k


def kernel(env_ids, env2sketchs, sketch_table):
    B, L = env_ids.shape
    E, K = env2sketchs.shape
    V, D = sketch_table.shape
    N = B * L
    ids = env_ids.reshape(-1).astype(jnp.int32)
    e2s = env2sketchs.reshape(-1).astype(jnp.int32)
    table = sketch_table.astype(jnp.float32)
    k = _sc_kernel(N, E, K, V, D, C=512, EPT=512 // K)
    out_flat, _ = k(ids, e2s, table)
    return out_flat.reshape(B, L, D)


# speedup vs baseline: 37.5691x; 37.5691x over previous
import functools

import jax
import jax.numpy as jnp
from jax import lax
from jax.experimental import pallas as pl
from jax.experimental.pallas import tpu as pltpu
from jax.experimental.pallas import tpu_sc as plsc

NC = 2
NS = 16
NW = NC * NS


def _sc_kernel(N, E, K, V, D, C, EPT):
    per_w = N // NW
    n_chunks = per_w // C
    E_pad = ((E + EPT - 1) // EPT) * EPT
    mesh = plsc.VectorSubcoreMesh(
        core_axis_name="c", subcore_axis_name="s",
        num_cores=NC, num_subcores=NS)

    @functools.partial(
        pl.kernel,
        mesh=mesh,
        out_type=[
            jax.ShapeDtypeStruct((N, D), jnp.float32),
            jax.ShapeDtypeStruct((E_pad, D), jnp.float32),
        ],
        scratch_types=[
            pltpu.VMEM((C,), jnp.int32),
            pltpu.VMEM((C, D), jnp.float32),
            pltpu.VMEM((EPT, D), jnp.float32),
            pltpu.SemaphoreType.DMA,
        ],
        compiler_params=pltpu.CompilerParams(use_tc_tiling_on_sc=False),
    )
    def k(ids_hbm, e2s_hbm, table_hbm, out_hbm, emb_hbm, idx_v, rows_v,
          emb_v, sem):
        c = lax.axis_index("c")
        s = lax.axis_index("s")
        wid = s * NC + c

        base = jnp.minimum(s * EPT, E - EPT)
        pltpu.sync_copy(e2s_hbm.at[pl.ds(base * K, EPT * K)], idx_v)
        pltpu.async_copy(table_hbm.at[idx_v], rows_v, sem).wait()

        def env_body(e, _):
            for d in range(D // 16):
                sl = pl.ds(d * 16, 16)
                acc = rows_v[e * K, sl]
                for j in range(1, K):
                    acc = acc + rows_v[e * K + j, sl]
                emb_v[e, sl] = acc
            return 0

        lax.fori_loop(0, EPT, env_body, 0)
        pltpu.sync_copy(emb_v, emb_hbm.at[pl.ds(base, EPT)])
        plsc.subcore_barrier()

        def chunk_body(i, _):
            off = wid * per_w + i * C
            pltpu.sync_copy(ids_hbm.at[pl.ds(off, C)], idx_v)
            pltpu.async_copy(emb_hbm.at[idx_v], rows_v, sem).wait()
            pltpu.sync_copy(rows_v, out_hbm.at[pl.ds(off, C)])
            return 0

        lax.fori_loop(0, n_chunks, chunk_body, 0)

    return k


def kernel(env_ids, env2sketchs, sketch_table):
    B, L = env_ids.shape
    E, K = env2sketchs.shape
    V, D = sketch_table.shape
    N = B * L
    ids = env_ids.reshape(-1).astype(jnp.int32)
    e2s = env2sketchs.reshape(-1).astype(jnp.int32)
    table = sketch_table.astype(jnp.float32)
    k = _sc_kernel(N, E, K, V, D, C=512, EPT=512 // K)
    out_flat, _ = k(ids, e2s, table)
    return out_flat.reshape(B, L, D)


# speedup vs baseline: 37.9425x; 1.0099x over previous
import functools

import jax
import jax.numpy as jnp
from jax import lax
from jax.experimental import pallas as pl
from jax.experimental.pallas import tpu as pltpu
from jax.experimental.pallas import tpu_sc as plsc

NC = 2
NS = 16
NW = NC * NS
NBUF = 4
LOOK = 2


def _sc_kernel(N, E, K, V, D, C, EPT):
    per_w = N // NW
    n_chunks = per_w // C
    assert per_w % C == 0 and n_chunks % NBUF == 0 and n_chunks >= 2 * NBUF
    E_pad = ((E + EPT - 1) // EPT) * EPT
    half = EPT // 2
    mesh = plsc.VectorSubcoreMesh(
        core_axis_name="c", subcore_axis_name="s",
        num_cores=NC, num_subcores=NS)

    @functools.partial(
        pl.kernel,
        mesh=mesh,
        out_type=[
            jax.ShapeDtypeStruct((N, D), jnp.float32),
            jax.ShapeDtypeStruct((E_pad, D), jnp.float32),
        ],
        scratch_types=[
            pltpu.VMEM((half * K,), jnp.int32),
            pltpu.VMEM((per_w,), jnp.int32),
            pltpu.VMEM((NBUF, C, D), jnp.float32),
            pltpu.VMEM((EPT, D), jnp.float32),
            pltpu.SemaphoreType.DMA,
            pltpu.SemaphoreType.DMA((NBUF,)),
            pltpu.SemaphoreType.DMA((NBUF,)),
        ],
        compiler_params=pltpu.CompilerParams(use_tc_tiling_on_sc=False),
    )
    def k(ids_hbm, e2s_hbm, table_hbm, out_hbm, emb_hbm, eidx_v, idx_all,
          rows, emb_v, isem, gsem, wsem):
        c = lax.axis_index("c")
        s = lax.axis_index("s")
        wid = s * NC + c
        woff = wid * per_w

        idcopy = pltpu.async_copy(ids_hbm.at[pl.ds(woff, per_w)], idx_all,
                                  isem)

        base = jnp.minimum(s * EPT, E - EPT)
        for h in range(2):
            hbase = base + h * half
            pltpu.sync_copy(e2s_hbm.at[pl.ds(hbase * K, half * K)], eidx_v)
            pltpu.async_copy(table_hbm.at[eidx_v],
                             rows.at[0].at[pl.ds(0, half * K)],
                             gsem.at[0]).wait()

            def env_body(e, _):
                for d in range(D // 16):
                    sl = pl.ds(d * 16, 16)
                    acc = rows[0, e * K, sl]
                    for j in range(1, K):
                        acc = acc + rows[0, e * K + j, sl]
                    emb_v[h * half + e, sl] = acc
                return 0

            lax.fori_loop(0, half, env_body, 0)
        pltpu.sync_copy(emb_v, emb_hbm.at[pl.ds(base, EPT)])
        plsc.subcore_barrier()
        idcopy.wait()

        def gidx(i):
            return idx_all.at[pl.ds(i * C, C)]

        def start_gather(i, b):
            return pltpu.async_copy(emb_hbm.at[gidx(i)], rows.at[b],
                                    gsem.at[b])

        def wait_gather(i, b):
            pltpu.make_async_copy(emb_hbm.at[gidx(i)], rows.at[b],
                                  gsem.at[b]).wait()

        def start_write(i, b):
            return pltpu.async_copy(rows.at[b],
                                    out_hbm.at[pl.ds(woff + i * C, C)],
                                    wsem.at[b])

        def wait_write(i, b):
            pltpu.make_async_copy(rows.at[b],
                                  out_hbm.at[pl.ds(woff + i * C, C)],
                                  wsem.at[b]).wait()

        start_gather(0, 0)
        start_gather(1, 1)
        for b in range(NBUF):
            i = b
            wait_gather(i, b)
            start_write(i, b)
            if b < LOOK:
                start_gather(i + LOOK, (b + LOOK) % NBUF)
            else:
                wait_write(i - LOOK, (b + LOOK) % NBUF)
                start_gather(i + LOOK, (b + LOOK) % NBUF)

        def group(g, _):
            for b in range(NBUF):
                i = g * NBUF + b
                wait_gather(i, b)
                start_write(i, b)
                j = i + LOOK
                bj = (b + LOOK) % NBUF
                wait_write(j - NBUF, bj)

                @pl.when(j < n_chunks)
                def _():
                    start_gather(j, bj)

            return 0

        lax.fori_loop(1, n_chunks // NBUF, group, 0)

        wait_write(n_chunks - 2, (n_chunks - 2) % NBUF)
        wait_write(n_chunks - 1, (n_chunks - 1) % NBUF)

    return k


def kernel(env_ids, env2sketchs, sketch_table):
    B, L = env_ids.shape
    E, K = env2sketchs.shape
    V, D = sketch_table.shape
    N = B * L
    ids = env_ids.reshape(-1).astype(jnp.int32)
    e2s = env2sketchs.reshape(-1).astype(jnp.int32)
    table = sketch_table.astype(jnp.float32)
    k = _sc_kernel(N, E, K, V, D, C=320, EPT=64)
    out_flat, _ = k(ids, e2s, table)
    return out_flat.reshape(B, L, D)
